# Initial kernel scaffold; baseline (speedup 1.0000x reference)
import jax
import jax.numpy as jnp
from jax.experimental import pallas as pl


def kernel(users, items, edge_index, edge_weight, user_emb, item_emb):
    raise NotImplementedError("write your pallas kernel here")


# speedup vs baseline: 1.0008x; 1.0008x over previous
import jax
import jax.numpy as jnp
from jax.experimental import pallas as pl

N_USERS = 25000
N_NODES = 50000
LAYERS = 3


def _dot_body(u_ref, i_ref, o_ref):
    o_ref[:] = jnp.sum(u_ref[:] * i_ref[:], axis=1)


def kernel(users, items, edge_index, edge_weight, user_emb, item_emb):
    all_emb = jnp.concatenate([user_emb, item_emb], axis=0)
    embs = [all_emb]
    row = edge_index[0]
    col = edge_index[1]
    for _ in range(LAYERS):
        msgs = jnp.take(all_emb, col, axis=0) * edge_weight[:, None]
        all_emb = jax.ops.segment_sum(msgs, row, num_segments=N_NODES)
        embs.append(all_emb)
    light_out = jnp.mean(jnp.stack(embs, axis=1), axis=1)
    users_emb = jnp.take(light_out[:N_USERS], users, axis=0)
    items_emb = jnp.take(light_out[N_USERS:], items, axis=0)
    gamma = pl.pallas_call(
        _dot_body,
        out_shape=jax.ShapeDtypeStruct((users_emb.shape[0],), jnp.float32),
    )(users_emb, items_emb)
    return gamma


# speedup vs baseline: 2.6457x; 2.6436x over previous
import functools

import jax
import jax.numpy as jnp
from jax import lax
from jax.experimental import pallas as pl
from jax.experimental.pallas import tpu as pltpu
from jax.experimental.pallas import tpu_sc as plsc

N_USERS = 25000
N_NODES = 50000
NP = 50048
HALF = 32
E = 800000
BATCH = 4096
NS = 16
EPT = E // NS
CHUNK = 80
NCHUNK = EPT // CHUNK
RPT = NP // NS
ZROWS = 184
DR = 184
BPT = BATCH // NS

_mesh = plsc.VectorSubcoreMesh(core_axis_name="c", subcore_axis_name="s")


@functools.partial(
    pl.kernel,
    mesh=_mesh,
    compiler_params=pltpu.CompilerParams(use_tc_tiling_on_sc=False),
    out_type=[
        jax.ShapeDtypeStruct((2 * NP, HALF), jnp.float32),
        jax.ShapeDtypeStruct((2 * NP, HALF), jnp.float32),
        jax.ShapeDtypeStruct((2 * NP, HALF), jnp.float32),
    ] + [jax.ShapeDtypeStruct((2 * BATCH, HALF), jnp.float32)] * 8,
    scratch_types=[
        pltpu.VMEM_SHARED((NP, HALF), jnp.float32),
        pltpu.VMEM((ZROWS, HALF), jnp.float32),
        pltpu.VMEM((DR, HALF), jnp.float32),
        pltpu.VMEM((CHUNK,), jnp.int32),
        pltpu.VMEM((CHUNK,), jnp.int32),
        pltpu.VMEM((CHUNK,), jnp.float32),
        pltpu.VMEM((CHUNK,), jnp.int32),
        pltpu.VMEM((CHUNK, HALF), jnp.float32),
        pltpu.VMEM((64,), jnp.int32),
        pltpu.VMEM((64,), jnp.int32),
        pltpu.VMEM((64, HALF), jnp.float32),
        pltpu.VMEM((64, HALF), jnp.float32),
        pltpu.VMEM((64, HALF), jnp.float32),
        pltpu.VMEM((64, HALF), jnp.float32),
        pltpu.SemaphoreType.DMA,
    ],
)
def _sc_lightgcn(t0, col_h, row_h, w_h, users_h, items_h,
                 t1, t2, t3, u0, u1, u2, u3, i0, i1, i2, i3,
                 acc, zbuf, dbuf, col_v, row_v, w_v, idx_v, msgs,
                 bidx, gidx, g0, g1, g2, g3, sem):
    c = lax.axis_index("c")
    s = lax.axis_index("s")
    coff = jnp.full((16,), c * NP, jnp.int32)
    z16 = jnp.zeros((16,), jnp.float32)

    def zb_init(r, carry):
        zbuf[r, pl.ds(0, 16)] = z16
        zbuf[r, pl.ds(16, 16)] = z16
        return carry

    lax.fori_loop(0, ZROWS, zb_init, 0)

    def do_layer(tin, tout):
        def zero_body(b, carry):
            pltpu.sync_copy(zbuf, acc.at[pl.ds(s * RPT + b * ZROWS, ZROWS)])
            return carry

        lax.fori_loop(0, RPT // ZROWS, zero_body, 0)
        plsc.subcore_barrier()

        def chunk_body(i, carry):
            base = s * EPT + i * CHUNK
            pltpu.sync_copy(col_h.at[pl.ds(base, CHUNK)], col_v)
            pltpu.sync_copy(row_h.at[pl.ds(base, CHUNK)], row_v)
            pltpu.sync_copy(w_h.at[pl.ds(base, CHUNK)], w_v)
            for j in range(CHUNK // 16):
                idx_v[pl.ds(j * 16, 16)] = col_v[pl.ds(j * 16, 16)] + coff
            pltpu.async_copy(tin.at[idx_v], msgs, sem).wait()
            for g in range(CHUNK // 16):
                w16 = w_v[pl.ds(g * 16, 16)]
                for t in range(16):
                    e = g * 16 + t
                    ws = w16[t]
                    msgs[e, pl.ds(0, 16)] = msgs[e, pl.ds(0, 16)] * ws
                    msgs[e, pl.ds(16, 16)] = msgs[e, pl.ds(16, 16)] * ws
            pltpu.sync_copy(msgs, acc.at[row_v], add=True)
            return carry

        lax.fori_loop(0, NCHUNK, chunk_body, 0)
        plsc.subcore_barrier()

        def drain_body(b, carry):
            r0 = s * RPT + b * DR
            pltpu.sync_copy(acc.at[pl.ds(r0, DR)], dbuf)
            pltpu.sync_copy(dbuf, tout.at[pl.ds(c * NP + r0, DR)])
            return carry

        lax.fori_loop(0, RPT // DR, drain_body, 0)
        plsc.subcore_barrier()

    do_layer(t0, t1)
    do_layer(t1, t2)
    do_layer(t2, t3)

    for h in range(BPT // 64):
        b0 = s * BPT + h * 64
        o0 = c * BATCH + b0

        pltpu.sync_copy(users_h.at[pl.ds(b0, 64)], bidx)
        for j in range(4):
            gidx[pl.ds(j * 16, 16)] = bidx[pl.ds(j * 16, 16)] + coff
        pltpu.async_copy(t0.at[gidx], g0, sem).wait()
        pltpu.async_copy(t1.at[gidx], g1, sem).wait()
        pltpu.async_copy(t2.at[gidx], g2, sem).wait()
        pltpu.async_copy(t3.at[gidx], g3, sem).wait()
        pltpu.sync_copy(g0, u0.at[pl.ds(o0, 64)])
        pltpu.sync_copy(g1, u1.at[pl.ds(o0, 64)])
        pltpu.sync_copy(g2, u2.at[pl.ds(o0, 64)])
        pltpu.sync_copy(g3, u3.at[pl.ds(o0, 64)])

        pltpu.sync_copy(items_h.at[pl.ds(b0, 64)], bidx)
        for j in range(4):
            gidx[pl.ds(j * 16, 16)] = bidx[pl.ds(j * 16, 16)] + coff + N_USERS
        pltpu.async_copy(t0.at[gidx], g0, sem).wait()
        pltpu.async_copy(t1.at[gidx], g1, sem).wait()
        pltpu.async_copy(t2.at[gidx], g2, sem).wait()
        pltpu.async_copy(t3.at[gidx], g3, sem).wait()
        pltpu.sync_copy(g0, i0.at[pl.ds(o0, 64)])
        pltpu.sync_copy(g1, i1.at[pl.ds(o0, 64)])
        pltpu.sync_copy(g2, i2.at[pl.ds(o0, 64)])
        pltpu.sync_copy(g3, i3.at[pl.ds(o0, 64)])


def _tc_dot_body(u0r, u1r, u2r, u3r, i0r, i1r, i2r, i3r, out_ref):
    us = u0r[...] + u1r[...] + u2r[...] + u3r[...]
    vs = i0r[...] + i1r[...] + i2r[...] + i3r[...]
    p = jnp.sum(us * vs, axis=1) * jnp.float32(1.0 / 16.0)
    out_ref[...] = p[:BATCH] + p[BATCH:]


def kernel(users, items, edge_index, edge_weight, user_emb, item_emb):
    all_emb = jnp.concatenate([user_emb, item_emb], axis=0)
    t0 = all_emb.reshape(N_NODES, 2, HALF).transpose(1, 0, 2).reshape(2 * N_NODES, HALF)
    t0 = jnp.pad(t0.reshape(2, N_NODES, HALF), ((0, 0), (0, NP - N_NODES), (0, 0))).reshape(2 * NP, HALF)
    row = edge_index[0].astype(jnp.int32)
    col = edge_index[1].astype(jnp.int32)
    users_i = users.astype(jnp.int32)
    items_i = items.astype(jnp.int32)
    outs = _sc_lightgcn(t0, col, row, edge_weight, users_i, items_i)
    gath = outs[3:]
    gamma = pl.pallas_call(
        _tc_dot_body,
        out_shape=jax.ShapeDtypeStruct((BATCH,), jnp.float32),
    )(*gath)
    return gamma


# speedup vs baseline: 8.4410x; 3.1904x over previous
import functools

import jax
import jax.numpy as jnp
from jax import lax
from jax.experimental import pallas as pl
from jax.experimental.pallas import tpu as pltpu
from jax.experimental.pallas import tpu_sc as plsc

N_USERS = 25000
N_NODES = 50000
NP = 50048
HALF = 32
E = 800000
BATCH = 4096
NS = 16
EPT = E // NS
CHUNK = 80
SUP = 2000
CPS = SUP // CHUNK
NSUP = EPT // SUP
RPT = NP // NS
ZROWS = 184
BPT = BATCH // NS

_mesh = plsc.VectorSubcoreMesh(core_axis_name="c", subcore_axis_name="s")


@functools.partial(
    pl.kernel,
    mesh=_mesh,
    compiler_params=pltpu.CompilerParams(use_tc_tiling_on_sc=False),
    out_type=[
        jax.ShapeDtypeStruct((2 * NP, HALF), jnp.float32),
        jax.ShapeDtypeStruct((2 * NP, HALF), jnp.float32),
        jax.ShapeDtypeStruct((2 * NP, HALF), jnp.float32),
    ] + [jax.ShapeDtypeStruct((2 * BATCH, HALF), jnp.float32)] * 8,
    scratch_types=[
        pltpu.VMEM_SHARED((NP, HALF), jnp.float32),
        pltpu.VMEM((ZROWS, HALF), jnp.float32),
        pltpu.VMEM((SUP,), jnp.int32),
        pltpu.VMEM((CPS, CHUNK), jnp.int32),
        pltpu.VMEM((SUP,), jnp.float32),
        pltpu.VMEM((SUP,), jnp.int32),
        pltpu.VMEM((CHUNK, HALF), jnp.float32),
        pltpu.VMEM((CHUNK, HALF), jnp.float32),
        pltpu.VMEM((64,), jnp.int32),
        pltpu.VMEM((64,), jnp.int32),
        pltpu.VMEM((64, HALF), jnp.float32),
        pltpu.VMEM((64, HALF), jnp.float32),
        pltpu.VMEM((64, HALF), jnp.float32),
        pltpu.VMEM((64, HALF), jnp.float32),
        pltpu.SemaphoreType.DMA,
        pltpu.SemaphoreType.DMA,
        pltpu.SemaphoreType.DMA,
        pltpu.SemaphoreType.DMA,
        pltpu.SemaphoreType.DMA,
    ],
)
def _sc_lightgcn(t0, col_h, row2d_h, w_h, users_h, items_h,
                 t1, t2, t3, u0, u1, u2, u3, i0, i1, i2, i3,
                 acc, zdbuf, col_big, rowb, w_big, idx_big, msgs0, msgs1,
                 bidx, gidx, g0, g1, g2, g3,
                 gsem0, gsem1, ssem0, ssem1, sem):
    c = lax.axis_index("c")
    s = lax.axis_index("s")
    coff = jnp.full((16,), c * NP, jnp.int32)
    z16 = jnp.zeros((16,), jnp.float32)

    def idx_slice(k):
        return idx_big.at[pl.ds(k * CHUNK, CHUNK)]

    def mul_chunk(m, k):
        def grp(gi, carry):
            w16 = w_big[pl.ds(k * CHUNK + gi * 16, 16)]
            for t in range(16):
                ws = w16[t]
                e = gi * 16 + t
                m[e, pl.ds(0, 16)] = m[e, pl.ds(0, 16)] * ws
                m[e, pl.ds(16, 16)] = m[e, pl.ds(16, 16)] * ws
            return carry

        lax.fori_loop(0, CHUNK // 16, grp, 0)

    def issue_gather(tin, k, m, gsem):
        pltpu.async_copy(tin.at[idx_slice(k)], m, gsem)

    def wait_gather(tin, k, m, gsem):
        pltpu.make_async_copy(tin.at[idx_slice(k)], m, gsem).wait()

    def issue_scatter(k, m, ssem):
        pltpu.async_copy(m, acc.at[rowb.at[k]], ssem, add=True)

    def wait_scatter(k, m, ssem):
        pltpu.make_async_copy(m, acc.at[rowb.at[k]], ssem).wait()

    def do_layer(tin, tout):
        def zb_init(r, carry):
            zdbuf[r, pl.ds(0, 16)] = z16
            zdbuf[r, pl.ds(16, 16)] = z16
            return carry

        lax.fori_loop(0, ZROWS, zb_init, 0)

        def zero_body(b, carry):
            pltpu.sync_copy(zdbuf, acc.at[pl.ds(s * RPT + b * ZROWS, ZROWS)])
            return carry

        lax.fori_loop(0, RPT // ZROWS, zero_body, 0)
        plsc.subcore_barrier()

        def sup_body(u, carry):
            base = s * EPT + u * SUP
            pltpu.sync_copy(col_h.at[pl.ds(base, SUP)], col_big)
            pltpu.sync_copy(w_h.at[pl.ds(base, SUP)], w_big)
            pltpu.sync_copy(row2d_h.at[pl.ds(s * (EPT // CHUNK) + u * CPS, CPS)],
                            rowb)

            def idx_body(j, carry2):
                idx_big[pl.ds(j * 16, 16)] = col_big[pl.ds(j * 16, 16)] + coff
                return carry2

            lax.fori_loop(0, SUP // 16, idx_body, 0)

            issue_gather(tin, 0, msgs0, gsem0)
            issue_gather(tin, 1, msgs1, gsem1)
            wait_gather(tin, 0, msgs0, gsem0)
            mul_chunk(msgs0, 0)
            issue_scatter(0, msgs0, ssem0)

            def pair_body(kk, carry2):
                k1 = 2 * kk + 1
                wait_scatter(k1 - 1, msgs0, ssem0)
                issue_gather(tin, k1 + 1, msgs0, gsem0)
                wait_gather(tin, k1, msgs1, gsem1)
                mul_chunk(msgs1, k1)
                issue_scatter(k1, msgs1, ssem1)

                k2 = k1 + 1
                wait_scatter(k2 - 1, msgs1, ssem1)
                issue_gather(tin, k2 + 1, msgs1, gsem1)
                wait_gather(tin, k2, msgs0, gsem0)
                mul_chunk(msgs0, k2)
                issue_scatter(k2, msgs0, ssem0)
                return carry2

            lax.fori_loop(0, (CPS - 3) // 2, pair_body, 0)

            wait_scatter(22, msgs0, ssem0)
            issue_gather(tin, 24, msgs0, gsem0)
            wait_gather(tin, 23, msgs1, gsem1)
            mul_chunk(msgs1, 23)
            issue_scatter(23, msgs1, ssem1)
            wait_scatter(23, msgs1, ssem1)
            wait_gather(tin, 24, msgs0, gsem0)
            mul_chunk(msgs0, 24)
            issue_scatter(24, msgs0, ssem0)
            wait_scatter(24, msgs0, ssem0)
            return carry

        lax.fori_loop(0, NSUP, sup_body, 0)
        plsc.subcore_barrier()

        def drain_body(b, carry):
            r0 = s * RPT + b * ZROWS
            pltpu.sync_copy(acc.at[pl.ds(r0, ZROWS)], zdbuf)
            pltpu.sync_copy(zdbuf, tout.at[pl.ds(c * NP + r0, ZROWS)])
            return carry

        lax.fori_loop(0, RPT // ZROWS, drain_body, 0)
        plsc.subcore_barrier()

    do_layer(t0, t1)
    do_layer(t1, t2)
    do_layer(t2, t3)

    for h in range(BPT // 64):
        b0 = s * BPT + h * 64
        o0 = c * BATCH + b0

        pltpu.sync_copy(users_h.at[pl.ds(b0, 64)], bidx)
        for j in range(4):
            gidx[pl.ds(j * 16, 16)] = bidx[pl.ds(j * 16, 16)] + coff
        pltpu.async_copy(t0.at[gidx], g0, sem).wait()
        pltpu.async_copy(t1.at[gidx], g1, sem).wait()
        pltpu.async_copy(t2.at[gidx], g2, sem).wait()
        pltpu.async_copy(t3.at[gidx], g3, sem).wait()
        pltpu.sync_copy(g0, u0.at[pl.ds(o0, 64)])
        pltpu.sync_copy(g1, u1.at[pl.ds(o0, 64)])
        pltpu.sync_copy(g2, u2.at[pl.ds(o0, 64)])
        pltpu.sync_copy(g3, u3.at[pl.ds(o0, 64)])

        pltpu.sync_copy(items_h.at[pl.ds(b0, 64)], bidx)
        for j in range(4):
            gidx[pl.ds(j * 16, 16)] = bidx[pl.ds(j * 16, 16)] + coff + N_USERS
        pltpu.async_copy(t0.at[gidx], g0, sem).wait()
        pltpu.async_copy(t1.at[gidx], g1, sem).wait()
        pltpu.async_copy(t2.at[gidx], g2, sem).wait()
        pltpu.async_copy(t3.at[gidx], g3, sem).wait()
        pltpu.sync_copy(g0, i0.at[pl.ds(o0, 64)])
        pltpu.sync_copy(g1, i1.at[pl.ds(o0, 64)])
        pltpu.sync_copy(g2, i2.at[pl.ds(o0, 64)])
        pltpu.sync_copy(g3, i3.at[pl.ds(o0, 64)])


def _tc_dot_body(u0r, u1r, u2r, u3r, i0r, i1r, i2r, i3r, out_ref):
    us = u0r[...] + u1r[...] + u2r[...] + u3r[...]
    vs = i0r[...] + i1r[...] + i2r[...] + i3r[...]
    p = jnp.sum(us * vs, axis=1) * jnp.float32(1.0 / 16.0)
    out_ref[...] = p[:BATCH] + p[BATCH:]


def kernel(users, items, edge_index, edge_weight, user_emb, item_emb):
    all_emb = jnp.concatenate([user_emb, item_emb], axis=0)
    t0 = all_emb.reshape(N_NODES, 2, HALF).transpose(1, 0, 2).reshape(2 * N_NODES, HALF)
    t0 = jnp.pad(t0.reshape(2, N_NODES, HALF), ((0, 0), (0, NP - N_NODES), (0, 0))).reshape(2 * NP, HALF)
    row = edge_index[0].astype(jnp.int32)
    col = edge_index[1].astype(jnp.int32)
    row2d = row.reshape(E // CHUNK, CHUNK)
    users_i = users.astype(jnp.int32)
    items_i = items.astype(jnp.int32)
    outs = _sc_lightgcn(t0, col, row2d, edge_weight, users_i, items_i)
    gath = outs[3:]
    gamma = pl.pallas_call(
        _tc_dot_body,
        out_shape=jax.ShapeDtypeStruct((BATCH,), jnp.float32),
    )(*gath)
    return gamma


# speedup vs baseline: 12.0439x; 1.4268x over previous
import functools

import jax
import jax.numpy as jnp
from jax import lax
from jax.experimental import pallas as pl
from jax.experimental.pallas import tpu as pltpu
from jax.experimental.pallas import tpu_sc as plsc

N_USERS = 25000
N_NODES = 50000
NP = 50048
HALF = 32
E = 800000
BATCH = 4096
NS = 16
EPT = E // NS
CHUNK = 80
SUP = 2000
CPS = SUP // CHUNK
NSUP = EPT // SUP
RPT = NP // NS
ZROWS = 184
BPT = BATCH // NS

_mesh = plsc.VectorSubcoreMesh(core_axis_name="c", subcore_axis_name="s")


@functools.partial(
    pl.kernel,
    mesh=_mesh,
    compiler_params=pltpu.CompilerParams(use_tc_tiling_on_sc=False),
    out_type=[
        jax.ShapeDtypeStruct((2 * NP, HALF), jnp.float32),
        jax.ShapeDtypeStruct((2 * NP, HALF), jnp.float32),
        jax.ShapeDtypeStruct((2 * NP, HALF), jnp.float32),
    ] + [jax.ShapeDtypeStruct((2 * BATCH, HALF), jnp.float32)] * 8,
    scratch_types=[
        pltpu.VMEM_SHARED((NP, HALF), jnp.float32),
        pltpu.VMEM((ZROWS, HALF), jnp.float32),
        pltpu.VMEM((CPS, CHUNK), jnp.int32),
        pltpu.VMEM((SUP,), jnp.float32),
        pltpu.VMEM((SUP,), jnp.int32),
        pltpu.VMEM((CHUNK, HALF), jnp.float32),
        pltpu.VMEM((CHUNK, HALF), jnp.float32),
        pltpu.VMEM((CHUNK, HALF), jnp.float32),
        pltpu.VMEM((CHUNK, HALF), jnp.float32),
        pltpu.VMEM((BPT,), jnp.int32),
        pltpu.VMEM((BPT,), jnp.int32),
        pltpu.VMEM((64, HALF), jnp.float32),
        pltpu.VMEM((64, HALF), jnp.float32),
        pltpu.VMEM((64, HALF), jnp.float32),
        pltpu.VMEM((64, HALF), jnp.float32),
        pltpu.SemaphoreType.DMA,
        pltpu.SemaphoreType.DMA,
        pltpu.SemaphoreType.DMA,
        pltpu.SemaphoreType.DMA,
        pltpu.SemaphoreType.DMA,
        pltpu.SemaphoreType.DMA,
        pltpu.SemaphoreType.DMA,
        pltpu.SemaphoreType.DMA,
        pltpu.SemaphoreType.DMA,
    ],
)
def _sc_lightgcn(t0, colx_h, row2d_h, w_h, usersx_h, itemsx_h,
                 t1, t2, t3, u0, u1, u2, u3, i0, i1, i2, i3,
                 acc, zdbuf, rowb, w_big, idx_big,
                 msgs0, msgs1, msgs2, msgs3, ubidx, ibidx, g0, g1, g2, g3,
                 gsem0, gsem1, gsem2, gsem3, ssem0, ssem1, ssem2, ssem3, sem):
    c = lax.axis_index("c")
    s = lax.axis_index("s")
    z16 = jnp.zeros((16,), jnp.float32)
    MBUF = (msgs0, msgs1, msgs2, msgs3)
    GSEM = (gsem0, gsem1, gsem2, gsem3)
    SSEM = (ssem0, ssem1, ssem2, ssem3)

    def idx_slice(k):
        return idx_big.at[pl.ds(k * CHUNK, CHUNK)]

    def mul_chunk(m, k):
        def grp(gi, carry):
            w16 = w_big[pl.ds(k * CHUNK + gi * 16, 16)]
            for t in range(16):
                ws = w16[t]
                e = gi * 16 + t
                m[e, pl.ds(0, 16)] = m[e, pl.ds(0, 16)] * ws
                m[e, pl.ds(16, 16)] = m[e, pl.ds(16, 16)] * ws
            return carry

        lax.fori_loop(0, CHUNK // 16, grp, 0)

    def issue_gather(tin, k, p):
        pltpu.async_copy(tin.at[idx_slice(k)], MBUF[p], GSEM[p])

    def wait_gather(tin, k, p):
        pltpu.make_async_copy(tin.at[idx_slice(k)], MBUF[p], GSEM[p]).wait()

    def issue_scatter(k, p):
        pltpu.async_copy(MBUF[p], acc.at[rowb.at[k]], SSEM[p], add=True)

    def wait_scatter(k, p):
        pltpu.make_async_copy(MBUF[p], acc.at[rowb.at[k]], SSEM[p]).wait()

    def do_layer(tin, tout):
        def zb_init(r, carry):
            zdbuf[r, pl.ds(0, 16)] = z16
            zdbuf[r, pl.ds(16, 16)] = z16
            return carry

        lax.fori_loop(0, ZROWS, zb_init, 0)

        def zero_body(b, carry):
            pltpu.sync_copy(zdbuf, acc.at[pl.ds(s * RPT + b * ZROWS, ZROWS)])
            return carry

        lax.fori_loop(0, RPT // ZROWS, zero_body, 0)
        plsc.subcore_barrier()

        def sup_body(u, carry):
            base = s * EPT + u * SUP
            pltpu.sync_copy(colx_h.at[pl.ds(c * E + base, SUP)], idx_big)
            pltpu.sync_copy(w_h.at[pl.ds(base, SUP)], w_big)
            pltpu.sync_copy(row2d_h.at[pl.ds(s * (EPT // CHUNK) + u * CPS, CPS)],
                            rowb)

            issue_gather(tin, 0, 0)
            issue_gather(tin, 1, 1)
            issue_gather(tin, 2, 2)
            wait_gather(tin, 0, 0)
            mul_chunk(msgs0, 0)
            issue_scatter(0, 0)
            issue_gather(tin, 3, 3)

            def quad_body(kk, carry2):
                for q in range(4):
                    k = 4 * kk + 1 + q
                    p = (1 + q) % 4
                    wait_gather(tin, k, p)
                    mul_chunk(MBUF[p], k)
                    issue_scatter(k, p)
                    wait_scatter(k - 1, (p + 3) % 4)
                    issue_gather(tin, k + 3, (p + 3) % 4)
                return carry2

            lax.fori_loop(0, 5, quad_body, 0)

            for k in (21, 22, 23, 24):
                p = k % 4
                wait_gather(tin, k, p)
                mul_chunk(MBUF[p], k)
                issue_scatter(k, p)
                if k == 21:
                    wait_scatter(20, (p + 3) % 4)
                    issue_gather(tin, 24, (p + 3) % 4)
            wait_scatter(21, 21 % 4)
            wait_scatter(22, 22 % 4)
            wait_scatter(23, 23 % 4)
            wait_scatter(24, 24 % 4)
            return carry

        lax.fori_loop(0, NSUP, sup_body, 0)
        plsc.subcore_barrier()

        def drain_body(b, carry):
            r0 = s * RPT + b * ZROWS
            pltpu.sync_copy(acc.at[pl.ds(r0, ZROWS)], zdbuf)
            pltpu.sync_copy(zdbuf, tout.at[pl.ds(c * NP + r0, ZROWS)])
            return carry

        lax.fori_loop(0, RPT // ZROWS, drain_body, 0)
        plsc.subcore_barrier()

    do_layer(t0, t1)
    do_layer(t1, t2)
    do_layer(t2, t3)

    pltpu.sync_copy(usersx_h.at[pl.ds(c * BATCH + s * BPT, BPT)], ubidx)
    pltpu.sync_copy(itemsx_h.at[pl.ds(c * BATCH + s * BPT, BPT)], ibidx)
    for (bx, o0a, o1a, o2a, o3a) in ((ubidx, u0, u1, u2, u3),
                                     (ibidx, i0, i1, i2, i3)):
        for h in range(BPT // 64):
            b0 = h * 64
            o0 = c * BATCH + s * BPT + b0
            bslice = bx.at[pl.ds(b0, 64)]
            pltpu.async_copy(t0.at[bslice], g0, gsem0)
            pltpu.async_copy(t1.at[bslice], g1, gsem1)
            pltpu.async_copy(t2.at[bslice], g2, gsem2)
            pltpu.async_copy(t3.at[bslice], g3, gsem3)
            pltpu.make_async_copy(t0.at[bslice], g0, gsem0).wait()
            pltpu.make_async_copy(t1.at[bslice], g1, gsem1).wait()
            pltpu.make_async_copy(t2.at[bslice], g2, gsem2).wait()
            pltpu.make_async_copy(t3.at[bslice], g3, gsem3).wait()
            pltpu.sync_copy(g0, o0a.at[pl.ds(o0, 64)])
            pltpu.sync_copy(g1, o1a.at[pl.ds(o0, 64)])
            pltpu.sync_copy(g2, o2a.at[pl.ds(o0, 64)])
            pltpu.sync_copy(g3, o3a.at[pl.ds(o0, 64)])


def _tc_dot_body(u0r, u1r, u2r, u3r, i0r, i1r, i2r, i3r, out_ref):
    us = u0r[...] + u1r[...] + u2r[...] + u3r[...]
    vs = i0r[...] + i1r[...] + i2r[...] + i3r[...]
    p = jnp.sum(us * vs, axis=1) * jnp.float32(1.0 / 16.0)
    out_ref[...] = p[:BATCH] + p[BATCH:]


def kernel(users, items, edge_index, edge_weight, user_emb, item_emb):
    all_emb = jnp.concatenate([user_emb, item_emb], axis=0)
    t0 = all_emb.reshape(N_NODES, 2, HALF).transpose(1, 0, 2).reshape(2 * N_NODES, HALF)
    t0 = jnp.pad(t0.reshape(2, N_NODES, HALF), ((0, 0), (0, NP - N_NODES), (0, 0))).reshape(2 * NP, HALF)
    row = edge_index[0].astype(jnp.int32)
    col = edge_index[1].astype(jnp.int32)
    row2d = row.reshape(E // CHUNK, CHUNK)
    colx = jnp.concatenate([col, col + NP])
    usersx = jnp.concatenate([users.astype(jnp.int32),
                              users.astype(jnp.int32) + NP])
    itemsx = jnp.concatenate([items.astype(jnp.int32) + N_USERS,
                              items.astype(jnp.int32) + N_USERS + NP])
    outs = _sc_lightgcn(t0, colx, row2d, edge_weight, usersx, itemsx)
    gath = outs[3:]
    gamma = pl.pallas_call(
        _tc_dot_body,
        out_shape=jax.ShapeDtypeStruct((BATCH,), jnp.float32),
    )(*gath)
    return gamma


# speedup vs baseline: 12.0898x; 1.0038x over previous
import functools

import jax
import jax.numpy as jnp
from jax import lax
from jax.experimental import pallas as pl
from jax.experimental.pallas import tpu as pltpu
from jax.experimental.pallas import tpu_sc as plsc

N_USERS = 25000
N_NODES = 50000
NP = 50048
HALF = 32
E = 800000
BATCH = 4096
NS = 16
EPT = E // NS
CHUNK = 80
SUP = 2000
CPS = SUP // CHUNK
NSUP = EPT // SUP
RPT = NP // NS
ZROWS = 184
BPT = BATCH // NS

_mesh = plsc.VectorSubcoreMesh(core_axis_name="c", subcore_axis_name="s")


@functools.partial(
    pl.kernel,
    mesh=_mesh,
    compiler_params=pltpu.CompilerParams(use_tc_tiling_on_sc=False),
    out_type=[
        jax.ShapeDtypeStruct((2 * NP, HALF), jnp.float32),
        jax.ShapeDtypeStruct((2 * NP, HALF), jnp.float32),
        jax.ShapeDtypeStruct((2 * NP, HALF), jnp.float32),
    ] + [jax.ShapeDtypeStruct((2 * BATCH, HALF), jnp.float32)] * 8,
    scratch_types=[
        pltpu.VMEM_SHARED((NP, HALF), jnp.float32),
        pltpu.VMEM((ZROWS, HALF), jnp.float32),
        pltpu.VMEM((CPS, CHUNK), jnp.int32),
        pltpu.VMEM((SUP,), jnp.float32),
        pltpu.VMEM((SUP,), jnp.int32),
        pltpu.VMEM((CHUNK, HALF), jnp.float32),
        pltpu.VMEM((CHUNK, HALF), jnp.float32),
        pltpu.VMEM((CHUNK, HALF), jnp.float32),
        pltpu.VMEM((CHUNK, HALF), jnp.float32),
        pltpu.VMEM((BPT,), jnp.int32),
        pltpu.VMEM((BPT,), jnp.int32),
        pltpu.VMEM((64, HALF), jnp.float32),
        pltpu.VMEM((64, HALF), jnp.float32),
        pltpu.VMEM((64, HALF), jnp.float32),
        pltpu.VMEM((64, HALF), jnp.float32),
        pltpu.SemaphoreType.DMA,
        pltpu.SemaphoreType.DMA,
        pltpu.SemaphoreType.DMA,
        pltpu.SemaphoreType.DMA,
        pltpu.SemaphoreType.DMA,
        pltpu.SemaphoreType.DMA,
        pltpu.SemaphoreType.DMA,
        pltpu.SemaphoreType.DMA,
        pltpu.SemaphoreType.DMA,
    ],
)
def _sc_lightgcn(t0, colx_h, row2d_h, w_h, usersx_h, itemsx_h,
                 t1, t2, t3, u0, u1, u2, u3, i0, i1, i2, i3,
                 acc, zdbuf, rowb, w_big, idx_big,
                 msgs0, msgs1, msgs2, msgs3, ubidx, ibidx, g0, g1, g2, g3,
                 gsem0, gsem1, gsem2, gsem3, ssem0, ssem1, ssem2, ssem3, sem):
    c = lax.axis_index("c")
    s = lax.axis_index("s")
    z16 = jnp.zeros((16,), jnp.float32)
    MBUF = (msgs0, msgs1, msgs2, msgs3)
    GSEM = (gsem0, gsem1, gsem2, gsem3)
    SSEM = (ssem0, ssem1, ssem2, ssem3)

    def idx_slice(k):
        return idx_big.at[pl.ds(k * CHUNK, CHUNK)]

    def mul_chunk(m, k):
        def grp(gi, carry):
            w16 = w_big[pl.ds(k * CHUNK + gi * 16, 16)]
            for t in range(16):
                ws = w16[t]
                e = gi * 16 + t
                m[e, pl.ds(0, 16)] = m[e, pl.ds(0, 16)] * ws
                m[e, pl.ds(16, 16)] = m[e, pl.ds(16, 16)] * ws
            return carry

        lax.fori_loop(0, CHUNK // 16, grp, 0)

    def issue_gather(tin, k, p):
        pltpu.async_copy(tin.at[idx_slice(k)], MBUF[p], GSEM[p])

    def wait_gather(tin, k, p):
        pltpu.make_async_copy(tin.at[idx_slice(k)], MBUF[p], GSEM[p]).wait()

    def issue_scatter(k, p):
        pltpu.async_copy(MBUF[p], acc.at[rowb.at[k]], SSEM[p], add=True)

    def wait_scatter(k, p):
        pltpu.make_async_copy(MBUF[p], acc.at[rowb.at[k]], SSEM[p]).wait()

    def do_layer(tin, tout, zero_next):
        def sup_body(u, carry):
            base = s * EPT + u * SUP
            pltpu.sync_copy(colx_h.at[pl.ds(c * E + base, SUP)], idx_big)
            pltpu.sync_copy(w_h.at[pl.ds(base, SUP)], w_big)
            pltpu.sync_copy(row2d_h.at[pl.ds(s * (EPT // CHUNK) + u * CPS, CPS)],
                            rowb)

            issue_gather(tin, 0, 0)
            issue_gather(tin, 1, 1)
            issue_gather(tin, 2, 2)
            wait_gather(tin, 0, 0)
            mul_chunk(msgs0, 0)
            issue_scatter(0, 0)
            issue_gather(tin, 3, 3)

            def quad_body(kk, carry2):
                for q in range(4):
                    k = 4 * kk + 1 + q
                    p = (1 + q) % 4
                    wait_gather(tin, k, p)
                    mul_chunk(MBUF[p], k)
                    issue_scatter(k, p)
                    wait_scatter(k - 1, (p + 3) % 4)
                    issue_gather(tin, k + 3, (p + 3) % 4)
                return carry2

            lax.fori_loop(0, 5, quad_body, 0)

            for k in (21, 22, 23, 24):
                p = k % 4
                wait_gather(tin, k, p)
                mul_chunk(MBUF[p], k)
                issue_scatter(k, p)
                if k == 21:
                    wait_scatter(20, (p + 3) % 4)
                    issue_gather(tin, 24, (p + 3) % 4)
            wait_scatter(21, 21 % 4)
            wait_scatter(22, 22 % 4)
            wait_scatter(23, 23 % 4)
            wait_scatter(24, 24 % 4)
            return carry

        lax.fori_loop(0, NSUP, sup_body, 0)
        plsc.subcore_barrier()

        def drain_body(b, carry):
            r0 = s * RPT + b * ZROWS
            pltpu.sync_copy(acc.at[pl.ds(r0, ZROWS)], zdbuf)
            pltpu.sync_copy(zdbuf, tout.at[pl.ds(c * NP + r0, ZROWS)])
            return carry

        lax.fori_loop(0, RPT // ZROWS, drain_body, 0)

        if zero_next:
            def zb_req(r, carry):
                zdbuf[r, pl.ds(0, 16)] = z16
                zdbuf[r, pl.ds(16, 16)] = z16
                return carry

            lax.fori_loop(0, ZROWS, zb_req, 0)

            def zero_issue(b, carry):
                r0 = s * RPT + b * ZROWS
                pltpu.async_copy(zdbuf, acc.at[pl.ds(r0, ZROWS)], sem)
                return carry

            lax.fori_loop(0, RPT // ZROWS, zero_issue, 0)

            def zero_wait(b, carry):
                r0 = s * RPT + b * ZROWS
                pltpu.make_async_copy(zdbuf, acc.at[pl.ds(r0, ZROWS)],
                                      sem).wait()
                return carry

            lax.fori_loop(0, RPT // ZROWS, zero_wait, 0)
        plsc.subcore_barrier()

    def zb_init(r, carry):
        zdbuf[r, pl.ds(0, 16)] = z16
        zdbuf[r, pl.ds(16, 16)] = z16
        return carry

    lax.fori_loop(0, ZROWS, zb_init, 0)

    def zero0_issue(b, carry):
        pltpu.async_copy(zdbuf, acc.at[pl.ds(s * RPT + b * ZROWS, ZROWS)], sem)
        return carry

    lax.fori_loop(0, RPT // ZROWS, zero0_issue, 0)

    def zero0_wait(b, carry):
        pltpu.make_async_copy(zdbuf, acc.at[pl.ds(s * RPT + b * ZROWS, ZROWS)],
                              sem).wait()
        return carry

    lax.fori_loop(0, RPT // ZROWS, zero0_wait, 0)
    plsc.subcore_barrier()

    do_layer(t0, t1, True)
    do_layer(t1, t2, True)
    do_layer(t2, t3, False)

    pltpu.sync_copy(usersx_h.at[pl.ds(c * BATCH + s * BPT, BPT)], ubidx)
    pltpu.sync_copy(itemsx_h.at[pl.ds(c * BATCH + s * BPT, BPT)], ibidx)
    for (bx, o0a, o1a, o2a, o3a) in ((ubidx, u0, u1, u2, u3),
                                     (ibidx, i0, i1, i2, i3)):
        for h in range(BPT // 64):
            b0 = h * 64
            o0 = c * BATCH + s * BPT + b0
            bslice = bx.at[pl.ds(b0, 64)]
            pltpu.async_copy(t0.at[bslice], g0, gsem0)
            pltpu.async_copy(t1.at[bslice], g1, gsem1)
            pltpu.async_copy(t2.at[bslice], g2, gsem2)
            pltpu.async_copy(t3.at[bslice], g3, gsem3)
            pltpu.make_async_copy(t0.at[bslice], g0, gsem0).wait()
            pltpu.make_async_copy(t1.at[bslice], g1, gsem1).wait()
            pltpu.make_async_copy(t2.at[bslice], g2, gsem2).wait()
            pltpu.make_async_copy(t3.at[bslice], g3, gsem3).wait()
            pltpu.sync_copy(g0, o0a.at[pl.ds(o0, 64)])
            pltpu.sync_copy(g1, o1a.at[pl.ds(o0, 64)])
            pltpu.sync_copy(g2, o2a.at[pl.ds(o0, 64)])
            pltpu.sync_copy(g3, o3a.at[pl.ds(o0, 64)])


def _tc_dot_body(u0r, u1r, u2r, u3r, i0r, i1r, i2r, i3r, out_ref):
    us = u0r[...] + u1r[...] + u2r[...] + u3r[...]
    vs = i0r[...] + i1r[...] + i2r[...] + i3r[...]
    p = jnp.sum(us * vs, axis=1) * jnp.float32(1.0 / 16.0)
    out_ref[...] = p[:BATCH] + p[BATCH:]


def kernel(users, items, edge_index, edge_weight, user_emb, item_emb):
    all_emb = jnp.concatenate([user_emb, item_emb], axis=0)
    t0 = all_emb.reshape(N_NODES, 2, HALF).transpose(1, 0, 2).reshape(2 * N_NODES, HALF)
    t0 = jnp.pad(t0.reshape(2, N_NODES, HALF), ((0, 0), (0, NP - N_NODES), (0, 0))).reshape(2 * NP, HALF)
    row = edge_index[0].astype(jnp.int32)
    col = edge_index[1].astype(jnp.int32)
    row2d = row.reshape(E // CHUNK, CHUNK)
    colx = jnp.concatenate([col, col + NP])
    usersx = jnp.concatenate([users.astype(jnp.int32),
                              users.astype(jnp.int32) + NP])
    itemsx = jnp.concatenate([items.astype(jnp.int32) + N_USERS,
                              items.astype(jnp.int32) + N_USERS + NP])
    outs = _sc_lightgcn(t0, colx, row2d, edge_weight, usersx, itemsx)
    gath = outs[3:]
    gamma = pl.pallas_call(
        _tc_dot_body,
        out_shape=jax.ShapeDtypeStruct((BATCH,), jnp.float32),
    )(*gath)
    return gamma


# speedup vs baseline: 14.4790x; 1.1976x over previous
import functools

import jax
import jax.numpy as jnp
from jax import lax
from jax.experimental import pallas as pl
from jax.experimental.pallas import tpu as pltpu
from jax.experimental.pallas import tpu_sc as plsc

N_USERS = 25000
N_NODES = 50000
NP = 50048
HALF = 32
E = 800000
BATCH = 4096
NS = 16
EPT = E // NS
CHUNK = 80
SUP = 2000
CPS = SUP // CHUNK
NSUP = EPT // SUP
RPT = NP // NS
ZROWS = 184
BPT = BATCH // NS

_mesh = plsc.VectorSubcoreMesh(core_axis_name="c", subcore_axis_name="s")


@functools.partial(
    pl.kernel,
    mesh=_mesh,
    compiler_params=pltpu.CompilerParams(use_tc_tiling_on_sc=False),
    out_type=[
        jax.ShapeDtypeStruct((2 * NP, HALF), jnp.float32),
        jax.ShapeDtypeStruct((2 * NP, HALF), jnp.float32),
        jax.ShapeDtypeStruct((2 * NP, HALF), jnp.float32),
    ] + [jax.ShapeDtypeStruct((2 * BATCH, HALF), jnp.float32)] * 8,
    scratch_types=[
        pltpu.VMEM_SHARED((NP, HALF), jnp.float32),
        pltpu.VMEM((ZROWS, HALF), jnp.float32),
        pltpu.VMEM((CPS, CHUNK), jnp.int32),
        pltpu.VMEM((SUP,), jnp.float32),
        pltpu.VMEM((SUP,), jnp.int32),
        pltpu.VMEM((CHUNK, HALF), jnp.float32),
        pltpu.VMEM((CHUNK, HALF), jnp.float32),
        pltpu.VMEM((CHUNK, HALF), jnp.float32),
        pltpu.VMEM((CHUNK, HALF), jnp.float32),
        pltpu.VMEM((CHUNK, HALF), jnp.float32),
        pltpu.VMEM((CHUNK, HALF), jnp.float32),
        pltpu.VMEM((BPT,), jnp.int32),
        pltpu.VMEM((BPT,), jnp.int32),
        pltpu.SemaphoreType.DMA,
        pltpu.SemaphoreType.DMA,
        pltpu.SemaphoreType.DMA,
        pltpu.SemaphoreType.DMA,
        pltpu.SemaphoreType.DMA,
        pltpu.SemaphoreType.DMA,
        pltpu.SemaphoreType.DMA,
        pltpu.SemaphoreType.DMA,
        pltpu.SemaphoreType.DMA,
        pltpu.SemaphoreType.DMA,
        pltpu.SemaphoreType.DMA,
        pltpu.SemaphoreType.DMA,
        pltpu.SemaphoreType.DMA,
    ],
)
def _sc_lightgcn(t0, colx_h, row2d_h, w_h, usersx_h, itemsx_h,
                 t1, t2, t3, u0, u1, u2, u3, i0, i1, i2, i3,
                 acc, zdbuf, rowb, w_big, idx_big,
                 msgs0, msgs1, msgs2, msgs3, msgs4, msgs5, ubidx, ibidx,
                 gsem0, gsem1, gsem2, gsem3, gsem4, gsem5,
                 ssem0, ssem1, ssem2, ssem3, ssem4, ssem5, sem):
    c = lax.axis_index("c")
    s = lax.axis_index("s")
    z16 = jnp.zeros((16,), jnp.float32)
    MBUF = (msgs0, msgs1, msgs2, msgs3, msgs4, msgs5)
    GSEM = (gsem0, gsem1, gsem2, gsem3, gsem4, gsem5)
    SSEM = (ssem0, ssem1, ssem2, ssem3, ssem4, ssem5)
    ND = 6

    def idx_slice(k):
        return idx_big.at[pl.ds(k * CHUNK, CHUNK)]

    def mul_chunk(m, k):
        def grp(gi, carry):
            w16 = w_big[pl.ds(k * CHUNK + gi * 16, 16)]
            for t in range(16):
                ws = w16[t]
                e = gi * 16 + t
                m[e, pl.ds(0, 16)] = m[e, pl.ds(0, 16)] * ws
                m[e, pl.ds(16, 16)] = m[e, pl.ds(16, 16)] * ws
            return carry

        lax.fori_loop(0, CHUNK // 16, grp, 0)

    def issue_gather(tin, k, p):
        pltpu.async_copy(tin.at[idx_slice(k)], MBUF[p], GSEM[p])

    def wait_gather(tin, k, p):
        pltpu.make_async_copy(tin.at[idx_slice(k)], MBUF[p], GSEM[p]).wait()

    def issue_scatter(k, p):
        pltpu.async_copy(MBUF[p], acc.at[rowb.at[k]], SSEM[p], add=True)

    def wait_scatter(k, p):
        pltpu.make_async_copy(MBUF[p], acc.at[rowb.at[k]], SSEM[p]).wait()

    def do_layer(tin, tout, zero_next):
        def sup_body(u, carry):
            base = s * EPT + u * SUP
            pltpu.sync_copy(colx_h.at[pl.ds(c * E + base, SUP)], idx_big)
            pltpu.sync_copy(w_h.at[pl.ds(base, SUP)], w_big)
            pltpu.sync_copy(row2d_h.at[pl.ds(s * (EPT // CHUNK) + u * CPS, CPS)],
                            rowb)

            for kp in range(5):
                issue_gather(tin, kp, kp)
            wait_gather(tin, 0, 0)
            mul_chunk(msgs0, 0)
            issue_scatter(0, 0)
            issue_gather(tin, 5, 5)

            def sext_body(kk, carry2):
                for q in range(6):
                    k = 6 * kk + 1 + q
                    p = (1 + q) % 6
                    wait_gather(tin, k, p)
                    mul_chunk(MBUF[p], k)
                    issue_scatter(k, p)
                    wait_scatter(k - 1, (p + 5) % 6)
                    issue_gather(tin, k + 5, (p + 5) % 6)
                return carry2

            lax.fori_loop(0, 3, sext_body, 0)

            for k in (19, 20, 21, 22, 23, 24):
                p = k % 6
                wait_gather(tin, k, p)
                mul_chunk(MBUF[p], k)
                issue_scatter(k, p)
                if k == 19:
                    wait_scatter(18, (p + 5) % 6)
                    issue_gather(tin, 24, (p + 5) % 6)
            for k in (19, 20, 21, 22, 23, 24):
                wait_scatter(k, k % 6)
            return carry

        lax.fori_loop(0, NSUP, sup_body, 0)
        plsc.subcore_barrier()

        def drain_body(b, carry):
            r0 = s * RPT + b * ZROWS
            pltpu.sync_copy(acc.at[pl.ds(r0, ZROWS)], zdbuf)
            pltpu.sync_copy(zdbuf, tout.at[pl.ds(c * NP + r0, ZROWS)])
            return carry

        lax.fori_loop(0, RPT // ZROWS, drain_body, 0)

        if zero_next:
            def zb_req(r, carry):
                zdbuf[r, pl.ds(0, 16)] = z16
                zdbuf[r, pl.ds(16, 16)] = z16
                return carry

            lax.fori_loop(0, ZROWS, zb_req, 0)

            def zero_issue(b, carry):
                r0 = s * RPT + b * ZROWS
                pltpu.async_copy(zdbuf, acc.at[pl.ds(r0, ZROWS)], sem)
                return carry

            lax.fori_loop(0, RPT // ZROWS, zero_issue, 0)

            def zero_wait(b, carry):
                r0 = s * RPT + b * ZROWS
                pltpu.make_async_copy(zdbuf, acc.at[pl.ds(r0, ZROWS)],
                                      sem).wait()
                return carry

            lax.fori_loop(0, RPT // ZROWS, zero_wait, 0)
        plsc.subcore_barrier()

    def zb_init(r, carry):
        zdbuf[r, pl.ds(0, 16)] = z16
        zdbuf[r, pl.ds(16, 16)] = z16
        return carry

    lax.fori_loop(0, ZROWS, zb_init, 0)

    def zero0_issue(b, carry):
        pltpu.async_copy(zdbuf, acc.at[pl.ds(s * RPT + b * ZROWS, ZROWS)], sem)
        return carry

    lax.fori_loop(0, RPT // ZROWS, zero0_issue, 0)

    def zero0_wait(b, carry):
        pltpu.make_async_copy(zdbuf, acc.at[pl.ds(s * RPT + b * ZROWS, ZROWS)],
                              sem).wait()
        return carry

    lax.fori_loop(0, RPT // ZROWS, zero0_wait, 0)
    plsc.subcore_barrier()

    do_layer(t0, t1, True)
    do_layer(t1, t2, True)
    do_layer(t2, t3, False)

    pltpu.sync_copy(usersx_h.at[pl.ds(c * BATCH + s * BPT, BPT)], ubidx)
    pltpu.sync_copy(itemsx_h.at[pl.ds(c * BATCH + s * BPT, BPT)], ibidx)
    for (bx, o0a, o1a, o2a, o3a) in ((ubidx, u0, u1, u2, u3),
                                     (ibidx, i0, i1, i2, i3)):
        for h in range(BPT // 64):
            b0 = h * 64
            o0 = c * BATCH + s * BPT + b0
            bslice = bx.at[pl.ds(b0, 64)]
            gb0 = msgs0.at[pl.ds(0, 64)]
            gb1 = msgs1.at[pl.ds(0, 64)]
            gb2 = msgs2.at[pl.ds(0, 64)]
            gb3 = msgs3.at[pl.ds(0, 64)]
            pltpu.async_copy(t0.at[bslice], gb0, gsem0)
            pltpu.async_copy(t1.at[bslice], gb1, gsem1)
            pltpu.async_copy(t2.at[bslice], gb2, gsem2)
            pltpu.async_copy(t3.at[bslice], gb3, gsem3)
            pltpu.make_async_copy(t0.at[bslice], gb0, gsem0).wait()
            pltpu.make_async_copy(t1.at[bslice], gb1, gsem1).wait()
            pltpu.make_async_copy(t2.at[bslice], gb2, gsem2).wait()
            pltpu.make_async_copy(t3.at[bslice], gb3, gsem3).wait()
            pltpu.sync_copy(gb0, o0a.at[pl.ds(o0, 64)])
            pltpu.sync_copy(gb1, o1a.at[pl.ds(o0, 64)])
            pltpu.sync_copy(gb2, o2a.at[pl.ds(o0, 64)])
            pltpu.sync_copy(gb3, o3a.at[pl.ds(o0, 64)])


def _tc_dot_body(u0r, u1r, u2r, u3r, i0r, i1r, i2r, i3r, out_ref):
    us = u0r[...] + u1r[...] + u2r[...] + u3r[...]
    vs = i0r[...] + i1r[...] + i2r[...] + i3r[...]
    p = jnp.sum(us * vs, axis=1) * jnp.float32(1.0 / 16.0)
    out_ref[...] = p[:BATCH] + p[BATCH:]


def kernel(users, items, edge_index, edge_weight, user_emb, item_emb):
    all_emb = jnp.concatenate([user_emb, item_emb], axis=0)
    t0 = all_emb.reshape(N_NODES, 2, HALF).transpose(1, 0, 2).reshape(2 * N_NODES, HALF)
    t0 = jnp.pad(t0.reshape(2, N_NODES, HALF), ((0, 0), (0, NP - N_NODES), (0, 0))).reshape(2 * NP, HALF)
    row = edge_index[0].astype(jnp.int32)
    col = edge_index[1].astype(jnp.int32)
    row2d = row.reshape(E // CHUNK, CHUNK)
    colx = jnp.concatenate([col, col + NP])
    usersx = jnp.concatenate([users.astype(jnp.int32),
                              users.astype(jnp.int32) + NP])
    itemsx = jnp.concatenate([items.astype(jnp.int32) + N_USERS,
                              items.astype(jnp.int32) + N_USERS + NP])
    outs = _sc_lightgcn(t0, colx, row2d, edge_weight, usersx, itemsx)
    gath = outs[3:]
    gamma = pl.pallas_call(
        _tc_dot_body,
        out_shape=jax.ShapeDtypeStruct((BATCH,), jnp.float32),
    )(*gath)
    return gamma


# speedup vs baseline: 16.0972x; 1.1118x over previous
import functools

import jax
import jax.numpy as jnp
from jax import lax
from jax.experimental import pallas as pl
from jax.experimental.pallas import tpu as pltpu
from jax.experimental.pallas import tpu_sc as plsc

N_USERS = 25000
N_NODES = 50000
NP = 50048
HALF = 32
E = 800000
BATCH = 4096
NS = 16
EPT = E // NS
CHUNK = 80
SUP = 2000
CPS = SUP // CHUNK
NSUP = EPT // SUP
RPT = NP // NS
ZROWS = 184
BPT = BATCH // NS

_mesh = plsc.VectorSubcoreMesh(core_axis_name="c", subcore_axis_name="s")


@functools.partial(
    pl.kernel,
    mesh=_mesh,
    compiler_params=pltpu.CompilerParams(use_tc_tiling_on_sc=False),
    out_type=[
        jax.ShapeDtypeStruct((2 * NP, HALF), jnp.float32),
        jax.ShapeDtypeStruct((2 * NP, HALF), jnp.float32),
        jax.ShapeDtypeStruct((2 * NP, HALF), jnp.float32),
    ] + [jax.ShapeDtypeStruct((2 * BATCH, HALF), jnp.float32)] * 8,
    scratch_types=[
        pltpu.VMEM_SHARED((NP, HALF), jnp.float32),
        pltpu.VMEM((ZROWS, HALF), jnp.float32),
        pltpu.VMEM((CPS, CHUNK), jnp.int32),
        pltpu.VMEM((SUP,), jnp.float32),
        pltpu.VMEM((SUP,), jnp.int32),
        pltpu.VMEM((CHUNK, HALF), jnp.float32),
        pltpu.VMEM((CHUNK, HALF), jnp.float32),
        pltpu.VMEM((CHUNK, HALF), jnp.float32),
        pltpu.VMEM((CHUNK, HALF), jnp.float32),
        pltpu.VMEM((CHUNK, HALF), jnp.float32),
        pltpu.VMEM((CHUNK, HALF), jnp.float32),
        pltpu.VMEM((BPT,), jnp.int32),
        pltpu.VMEM((BPT,), jnp.int32),
        pltpu.SemaphoreType.DMA,
        pltpu.SemaphoreType.DMA,
        pltpu.SemaphoreType.DMA,
        pltpu.SemaphoreType.DMA,
        pltpu.SemaphoreType.DMA,
        pltpu.SemaphoreType.DMA,
        pltpu.SemaphoreType.DMA,
        pltpu.SemaphoreType.DMA,
        pltpu.SemaphoreType.DMA,
        pltpu.SemaphoreType.DMA,
        pltpu.SemaphoreType.DMA,
        pltpu.SemaphoreType.DMA,
        pltpu.SemaphoreType.DMA,
    ],
)
def _sc_lightgcn(t0, colx_h, row2d_h, w_h, usersx_h, itemsx_h,
                 t1, t2, t3, u0, u1, u2, u3, i0, i1, i2, i3,
                 acc, zdbuf, rowb, w_big, idx_big,
                 msgs0, msgs1, msgs2, msgs3, msgs4, msgs5, ubidx, ibidx,
                 gsem0, gsem1, gsem2, gsem3, gsem4, gsem5,
                 ssem0, ssem1, ssem2, ssem3, ssem4, ssem5, sem):
    c = lax.axis_index("c")
    s = lax.axis_index("s")
    z16 = jnp.zeros((16,), jnp.float32)
    MBUF = (msgs0, msgs1, msgs2, msgs3, msgs4, msgs5)
    GSEM = (gsem0, gsem1, gsem2, gsem3, gsem4, gsem5)
    SSEM = (ssem0, ssem1, ssem2, ssem3, ssem4, ssem5)
    ND = 6

    def idx_slice(k):
        return idx_big.at[pl.ds(k * CHUNK, CHUNK)]

    def mul_chunk(m, k):
        def grp(gi, carry):
            w16 = w_big[pl.ds(k * CHUNK + gi * 16, 16)]
            for t in range(16):
                ws = w16[t]
                e = gi * 16 + t
                m[e, pl.ds(0, 16)] = m[e, pl.ds(0, 16)] * ws
                m[e, pl.ds(16, 16)] = m[e, pl.ds(16, 16)] * ws
            return carry

        lax.fori_loop(0, CHUNK // 16, grp, 0)

    def issue_gather(tin, k, p):
        pltpu.async_copy(tin.at[idx_slice(k)], MBUF[p], GSEM[p])

    def wait_gather(tin, k, p):
        pltpu.make_async_copy(tin.at[idx_slice(k)], MBUF[p], GSEM[p]).wait()

    def issue_scatter(k, p):
        pltpu.async_copy(MBUF[p], acc.at[rowb.at[k]], SSEM[p], add=True)

    def wait_scatter(k, p):
        pltpu.make_async_copy(MBUF[p], acc.at[rowb.at[k]], SSEM[p]).wait()

    def do_layer(tin, tout, zero_next):
        def sup_body(u, carry):
            base = s * EPT + u * SUP
            rb = s * (EPT // CHUNK) + u * CPS
            pltpu.async_copy(colx_h.at[pl.ds(c * E + base, SUP)], idx_big, gsem0)
            pltpu.async_copy(w_h.at[pl.ds(base, SUP)], w_big, gsem1)
            pltpu.async_copy(row2d_h.at[pl.ds(rb, CPS)], rowb, gsem2)
            pltpu.make_async_copy(colx_h.at[pl.ds(c * E + base, SUP)], idx_big,
                                  gsem0).wait()
            pltpu.make_async_copy(w_h.at[pl.ds(base, SUP)], w_big,
                                  gsem1).wait()
            pltpu.make_async_copy(row2d_h.at[pl.ds(rb, CPS)], rowb,
                                  gsem2).wait()

            for kp in range(5):
                issue_gather(tin, kp, kp)
            wait_gather(tin, 0, 0)
            mul_chunk(msgs0, 0)
            issue_scatter(0, 0)
            issue_gather(tin, 5, 5)

            def sext_body(kk, carry2):
                for q in range(6):
                    k = 6 * kk + 1 + q
                    p = (1 + q) % 6
                    wait_gather(tin, k, p)
                    mul_chunk(MBUF[p], k)
                    issue_scatter(k, p)
                    wait_scatter(k - 1, (p + 5) % 6)
                    issue_gather(tin, k + 5, (p + 5) % 6)
                return carry2

            lax.fori_loop(0, 3, sext_body, 0)

            for k in (19, 20, 21, 22, 23, 24):
                p = k % 6
                wait_gather(tin, k, p)
                mul_chunk(MBUF[p], k)
                issue_scatter(k, p)
                if k == 19:
                    wait_scatter(18, (p + 5) % 6)
                    issue_gather(tin, 24, (p + 5) % 6)
            for k in (19, 20, 21, 22, 23, 24):
                wait_scatter(k, k % 6)
            return carry

        lax.fori_loop(0, NSUP, sup_body, 0)
        plsc.subcore_barrier()

        def drain_body(b, carry):
            r0 = s * RPT + b * ZROWS
            pltpu.sync_copy(acc.at[pl.ds(r0, ZROWS)], zdbuf)
            pltpu.sync_copy(zdbuf, tout.at[pl.ds(c * NP + r0, ZROWS)])
            return carry

        lax.fori_loop(0, RPT // ZROWS, drain_body, 0)

        if zero_next:
            def zb_req(r, carry):
                zdbuf[r, pl.ds(0, 16)] = z16
                zdbuf[r, pl.ds(16, 16)] = z16
                return carry

            lax.fori_loop(0, ZROWS, zb_req, 0)

            def zero_issue(b, carry):
                r0 = s * RPT + b * ZROWS
                pltpu.async_copy(zdbuf, acc.at[pl.ds(r0, ZROWS)], sem)
                return carry

            lax.fori_loop(0, RPT // ZROWS, zero_issue, 0)

            def zero_wait(b, carry):
                r0 = s * RPT + b * ZROWS
                pltpu.make_async_copy(zdbuf, acc.at[pl.ds(r0, ZROWS)],
                                      sem).wait()
                return carry

            lax.fori_loop(0, RPT // ZROWS, zero_wait, 0)
        plsc.subcore_barrier()

    def zb_init(r, carry):
        zdbuf[r, pl.ds(0, 16)] = z16
        zdbuf[r, pl.ds(16, 16)] = z16
        return carry

    lax.fori_loop(0, ZROWS, zb_init, 0)

    def zero0_issue(b, carry):
        pltpu.async_copy(zdbuf, acc.at[pl.ds(s * RPT + b * ZROWS, ZROWS)], sem)
        return carry

    lax.fori_loop(0, RPT // ZROWS, zero0_issue, 0)

    def zero0_wait(b, carry):
        pltpu.make_async_copy(zdbuf, acc.at[pl.ds(s * RPT + b * ZROWS, ZROWS)],
                              sem).wait()
        return carry

    lax.fori_loop(0, RPT // ZROWS, zero0_wait, 0)
    plsc.subcore_barrier()

    do_layer(t0, t1, True)
    do_layer(t1, t2, True)
    do_layer(t2, t3, False)

    pltpu.sync_copy(usersx_h.at[pl.ds(c * BATCH + s * BPT, BPT)], ubidx)
    pltpu.sync_copy(itemsx_h.at[pl.ds(c * BATCH + s * BPT, BPT)], ibidx)
    for (bx, o0a, o1a, o2a, o3a) in ((ubidx, u0, u1, u2, u3),
                                     (ibidx, i0, i1, i2, i3)):
        for h in range(BPT // 64):
            b0 = h * 64
            o0 = c * BATCH + s * BPT + b0
            bslice = bx.at[pl.ds(b0, 64)]
            gb0 = msgs0.at[pl.ds(0, 64)]
            gb1 = msgs1.at[pl.ds(0, 64)]
            gb2 = msgs2.at[pl.ds(0, 64)]
            gb3 = msgs3.at[pl.ds(0, 64)]
            pltpu.async_copy(t0.at[bslice], gb0, gsem0)
            pltpu.async_copy(t1.at[bslice], gb1, gsem1)
            pltpu.async_copy(t2.at[bslice], gb2, gsem2)
            pltpu.async_copy(t3.at[bslice], gb3, gsem3)
            pltpu.make_async_copy(t0.at[bslice], gb0, gsem0).wait()
            pltpu.make_async_copy(t1.at[bslice], gb1, gsem1).wait()
            pltpu.make_async_copy(t2.at[bslice], gb2, gsem2).wait()
            pltpu.make_async_copy(t3.at[bslice], gb3, gsem3).wait()
            pltpu.sync_copy(gb0, o0a.at[pl.ds(o0, 64)])
            pltpu.sync_copy(gb1, o1a.at[pl.ds(o0, 64)])
            pltpu.sync_copy(gb2, o2a.at[pl.ds(o0, 64)])
            pltpu.sync_copy(gb3, o3a.at[pl.ds(o0, 64)])


def _tc_dot_body(u0r, u1r, u2r, u3r, i0r, i1r, i2r, i3r, out_ref):
    us = u0r[...] + u1r[...] + u2r[...] + u3r[...]
    vs = i0r[...] + i1r[...] + i2r[...] + i3r[...]
    p = jnp.sum(us * vs, axis=1) * jnp.float32(1.0 / 16.0)
    out_ref[...] = p[:BATCH] + p[BATCH:]


def kernel(users, items, edge_index, edge_weight, user_emb, item_emb):
    all_emb = jnp.concatenate([user_emb, item_emb], axis=0)
    t0 = all_emb.reshape(N_NODES, 2, HALF).transpose(1, 0, 2).reshape(2 * N_NODES, HALF)
    t0 = jnp.pad(t0.reshape(2, N_NODES, HALF), ((0, 0), (0, NP - N_NODES), (0, 0))).reshape(2 * NP, HALF)
    row = edge_index[0].astype(jnp.int32)
    col = edge_index[1].astype(jnp.int32)
    row2d = row.reshape(E // CHUNK, CHUNK)
    colx = jnp.concatenate([col, col + NP])
    usersx = jnp.concatenate([users.astype(jnp.int32),
                              users.astype(jnp.int32) + NP])
    itemsx = jnp.concatenate([items.astype(jnp.int32) + N_USERS,
                              items.astype(jnp.int32) + N_USERS + NP])
    outs = _sc_lightgcn(t0, colx, row2d, edge_weight, usersx, itemsx)
    gath = outs[3:]
    gamma = pl.pallas_call(
        _tc_dot_body,
        out_shape=jax.ShapeDtypeStruct((BATCH,), jnp.float32),
    )(*gath)
    return gamma
